# Initial kernel scaffold; baseline (speedup 1.0000x reference)
import jax
import jax.numpy as jnp
from jax.experimental import pallas as pl


def kernel(x):
    raise NotImplementedError("write your pallas kernel here")


# speedup vs baseline: 8.0812x; 8.0812x over previous
import functools

import jax
import jax.numpy as jnp
from jax import lax
from jax.experimental import pallas as pl
from jax.experimental.pallas import tpu as pltpu
from jax.experimental.pallas import tpu_sc as plsc

B = 16384
N = 200
NCONT = 3
NW = 32
LANES = 16
ROWS_PER_W = B // NW
N_GROUPS = ROWS_PER_W // LANES


def _sev_body(x_hbm, out_hbm, buf, outb):
    nc = 2
    wid = lax.axis_index("s") * nc + lax.axis_index("c")
    base_row = wid * ROWS_PER_W

    lane = lax.iota(jnp.int32, LANES)
    lane_off = lane * N
    zeros = jnp.zeros((LANES,), jnp.float32)
    col = [jnp.full((LANES,), k, jnp.int32) for k in range(5)]

    for g in range(N_GROUPS):
        row0 = base_row + g * LANES
        dp0 = jnp.ones((LANES,), jnp.float32)
        dp1 = zeros
        dp2 = zeros
        dp3 = zeros
        dp4 = zeros
        for c in range(NCONT):
            pltpu.sync_copy(
                x_hbm.at[pl.ds(c * (B * N) + row0 * N, LANES * N)], buf)

            def step(i, carry):
                d0, d1, d2, d3, d4 = carry
                pi = plsc.load_gather(buf, [lane_off + i])
                om = 1.0 - pi
                n4 = d4 * om + d3 * pi
                n3 = d3 * om + d2 * pi
                n2 = d2 * om + d1 * pi
                n1 = d1 * om + d0 * pi
                n0 = d0 * om
                return n0, n1, n2, n3, n4

            dp0, dp1, dp2, dp3, dp4 = lax.fori_loop(
                0, N, step, (dp0, dp1, dp2, dp3, dp4))

        sev0 = dp0
        sev1 = dp1 + dp2
        sev2 = dp3 + dp4
        sev3 = 1.0 - (sev0 + sev1 + sev2)
        rows = g * LANES + lane
        for k, val in enumerate((sev0, sev1, sev2, sev3, zeros)):
            plsc.store_scatter(outb, [rows, col[k]], val)

    pltpu.sync_copy(outb, out_hbm.at[pl.ds(base_row, ROWS_PER_W)])


@jax.jit
def kernel(x):
    x_flat = x.reshape(-1)
    mesh = plsc.VectorSubcoreMesh(core_axis_name="c", subcore_axis_name="s")
    run = functools.partial(
        pl.kernel,
        mesh=mesh,
        out_type=jax.ShapeDtypeStruct((B, 5), jnp.float32),
        scratch_types=[
            pltpu.VMEM((LANES * N,), jnp.float32),
            pltpu.VMEM((ROWS_PER_W, 5), jnp.float32),
        ],
        compiler_params=pltpu.CompilerParams(needs_layout_passes=False),
    )(_sev_body)
    return run(x_flat)


# speedup vs baseline: 10.5085x; 1.3004x over previous
import functools

import jax
import jax.numpy as jnp
from jax import lax
from jax.experimental import pallas as pl
from jax.experimental.pallas import tpu as pltpu
from jax.experimental.pallas import tpu_sc as plsc

B = 16384
N = 200
NCONT = 3
NTOT = N * NCONT
NW = 32
LANES = 16
GROUP = 2 * LANES
ROWS_PER_W = B // NW
N_GROUPS = ROWS_PER_W // GROUP
UNROLL = 8


def _dp_steps(buf, row_idx, colv, dp):
    new = []
    for h in range(2):
        d0, d1, d2, d3, d4 = dp[h]
        pi = plsc.load_gather(buf, [row_idx[h], colv])
        om = 1.0 - pi
        n4 = d4 * om + d3 * pi
        n3 = d3 * om + d2 * pi
        n2 = d2 * om + d1 * pi
        n1 = d1 * om + d0 * pi
        n0 = d0 * om
        new.append((n0, n1, n2, n3, n4))
    return new


def _sev_body(x_hbm, out_hbm, buf_a, buf_b, outb, sem_a, sem_b):
    nc = 2
    wid = lax.axis_index("s") * nc + lax.axis_index("c")
    base_row = wid * ROWS_PER_W

    lane = lax.iota(jnp.int32, LANES)
    zeros = jnp.zeros((LANES,), jnp.float32)
    ones = jnp.ones((LANES,), jnp.float32)
    row_idx = [h * LANES + lane for h in range(2)]
    col_of = [jnp.full((LANES,), k, jnp.int32) for k in range(5)]

    bufs = (buf_a, buf_b)
    sems = (sem_a, sem_b)

    def start_fetch(g):
        row0 = base_row + g * GROUP
        slot = g % 2
        return [
            pltpu.async_copy(
                x_hbm.at[c, pl.ds(row0, GROUP), :],
                bufs[slot].at[:, pl.ds(c * N, N)],
                sems[slot],
            )
            for c in range(NCONT)
        ]

    pending = start_fetch(0)
    for g in range(N_GROUPS):
        if g + 1 < N_GROUPS:
            nxt = start_fetch(g + 1)
        for h in pending:
            h.wait()
        buf = bufs[g % 2]

        dp = [(ones, zeros, zeros, zeros, zeros) for _ in range(2)]

        def body(t, carry):
            colv = carry[0]
            dp = [carry[1:6], carry[6:11]]
            for _ in range(UNROLL):
                dp = _dp_steps(buf, row_idx, colv, dp)
                colv = colv + 1
            return (colv, *dp[0], *dp[1])

        colv0 = jnp.zeros((LANES,), jnp.int32)
        res = lax.fori_loop(
            0, NTOT // UNROLL, body, (colv0, *dp[0], *dp[1]))
        dp = [res[1:6], res[6:11]]

        for h in range(2):
            d0, d1, d2, d3, d4 = dp[h]
            sev0 = d0
            sev1 = d1 + d2
            sev2 = d3 + d4
            sev3 = 1.0 - (sev0 + sev1 + sev2)
            rows = g * GROUP + h * LANES + lane
            for k, val in enumerate((sev0, sev1, sev2, sev3, zeros)):
                plsc.store_scatter(outb, [rows, col_of[k]], val)

        if g + 1 < N_GROUPS:
            pending = nxt

    pltpu.sync_copy(outb, out_hbm.at[pl.ds(base_row, ROWS_PER_W)])


@jax.jit
def kernel(x):
    mesh = plsc.VectorSubcoreMesh(core_axis_name="c", subcore_axis_name="s")
    run = functools.partial(
        pl.kernel,
        mesh=mesh,
        out_type=jax.ShapeDtypeStruct((B, 5), jnp.float32),
        scratch_types=[
            pltpu.VMEM((GROUP, NTOT), jnp.float32),
            pltpu.VMEM((GROUP, NTOT), jnp.float32),
            pltpu.VMEM((ROWS_PER_W, 5), jnp.float32),
            pltpu.SemaphoreType.DMA,
            pltpu.SemaphoreType.DMA,
        ],
        compiler_params=pltpu.CompilerParams(
            needs_layout_passes=False, use_tc_tiling_on_sc=False),
    )(_sev_body)
    return run(x)
